# Initial kernel scaffold; baseline (speedup 1.0000x reference)
import jax
import jax.numpy as jnp
from jax.experimental import pallas as pl


def kernel(x, edge_index, edge_attr, meta_data, hn, cn, W0, We0, asrc0, adst0, aedge0, bias0, W1, We1, asrc1, adst1, aedge1, bias1, W2, We2, asrc2, adst2, aedge2, bias2, sp_w1, sp_b1, sp_w2, sp_b2, me_w, me_b, fc_w, fc_b, l0_wi, l0_wh, l0_bi, l0_bh, l1_wi, l1_wh, l1_bi, l1_bh, out_w, out_b):
    raise NotImplementedError("write your pallas kernel here")


# speedup vs baseline: 51.3954x; 51.3954x over previous
import functools

import jax
import jax.numpy as jnp
from jax import lax
from jax.experimental import pallas as pl
from jax.experimental.pallas import tpu as pltpu
from jax.experimental.pallas import tpu_sc as plsc

HEADS = 3
NC = 2
NS = 16
NW = NC * NS


def _ale_body(ocs, ea, We0, ae0, We1, ae1, We2, ae2, o0, o1, o2):
    ea_blk = ea[...]
    eb = ea_blk.shape[0]
    for (We, ae, o, oc) in ((We0, ae0, o0, ocs[0]),
                            (We1, ae1, o1, ocs[1]),
                            (We2, ae2, o2, ocs[2])):
        cols = []
        for t in range(HEADS):
            m_t = (We[:, t * oc:(t + 1) * oc] * ae[t, :][None, :]).sum(
                axis=1, keepdims=True)
            cols.append(jnp.dot(ea_blk, m_t, preferred_element_type=jnp.float32))
        o[...] = jnp.concatenate(
            cols + [jnp.zeros((eb, 1), jnp.float32)], axis=1)


def _make_ale_kernel(E, Eb):
    grid = (E // Eb,)
    full = lambda shape: pl.BlockSpec(shape, lambda i: (0,) * len(shape))
    return pl.pallas_call(
        functools.partial(_ale_body, (16, 16, 8)),
        grid=grid,
        in_specs=[
            pl.BlockSpec((Eb, 2), lambda i: (i, 0)),
            full((2, 48)), full((3, 16)),
            full((2, 48)), full((3, 16)),
            full((2, 24)), full((3, 8)),
        ],
        out_specs=[pl.BlockSpec((Eb, 4), lambda i: (i, 0))] * 3,
        out_shape=[jax.ShapeDtypeStruct((E, 4), jnp.float32)] * 3,
    )


def _node_proj(h, a_s, a_d, oc, nb):
    hts, als_cols, ald_cols = [], [], []
    for t in range(HEADS):
        ht = h[:, t * oc:(t + 1) * oc]
        if oc < 16:
            hts.append(jnp.concatenate(
                [ht, jnp.zeros((nb, 16 - oc), jnp.float32)], axis=1))
        else:
            hts.append(ht)
        als_cols.append((ht * a_s[t, :][None, :]).sum(axis=1, keepdims=True))
        ald_cols.append((ht * a_d[t, :][None, :]).sum(axis=1, keepdims=True))
    z13 = jnp.zeros((nb, 13), jnp.float32)
    als = jnp.concatenate(als_cols + [z13], axis=1)
    ald = jnp.concatenate(ald_cols + [z13], axis=1)
    return hts, als, ald


def _kx0_body(x, W, a_s, a_d, h0, h1, h2, als, ald):
    xb = x[...]
    h = jnp.dot(xb, W[...], preferred_element_type=jnp.float32)
    hts, als_b, ald_b = _node_proj(h, a_s[...], a_d[...], 16, xb.shape[0])
    h0[...], h1[...], h2[...] = hts
    als[...], ald[...] = als_b, ald_b


def _make_kx0_kernel(N, Nb):
    grid = (N // Nb,)
    full = lambda shape: pl.BlockSpec(shape, lambda i: (0,) * len(shape))
    row = lambda w: pl.BlockSpec((Nb, w), lambda i: (i, 0))
    return pl.pallas_call(
        _kx0_body,
        grid=grid,
        in_specs=[row(3), full((3, 48)), full((3, 16)), full((3, 16))],
        out_specs=[row(16)] * 5,
        out_shape=[jax.ShapeDtypeStruct((N, 16), jnp.float32)] * 5,
    )


def _gat_epilogue(num, den, bias, oc_prev):
    outs = []
    for t in range(HEADS):
        numt = (num[t, 0] + num[t, 1])[:, :oc_prev]
        dent = (den[0, :, t] + den[1, :, t])[:, None]
        g = jnp.where(dent > 0.0, numt / dent, 0.0)
        g = g + bias[0, t * oc_prev:(t + 1) * oc_prev][None, :]
        outs.append(g)
    x = jnp.concatenate(outs, axis=1)
    return jnp.where(x > 0.0, x, jnp.exp(x) - 1.0)


def _kx12_body(oc_prev, oc, num, den, bias, W, a_s, a_d,
               h0, h1, h2, als, ald):
    x = _gat_epilogue(num[...], den[...], bias[...], oc_prev)
    h = jnp.dot(x, W[...], preferred_element_type=jnp.float32)
    hts, als_b, ald_b = _node_proj(h, a_s[...], a_d[...], oc, x.shape[0])
    h0[...], h1[...], h2[...] = hts
    als[...], ald[...] = als_b, ald_b


def _make_kx12_kernel(N, Nb, oc_prev, oc, din):
    grid = (N // Nb,)
    full = lambda shape: pl.BlockSpec(shape, lambda i: (0,) * len(shape))
    row = lambda w: pl.BlockSpec((Nb, w), lambda i: (i, 0))
    num_spec = pl.BlockSpec((3, NC, Nb, 16), lambda i: (0, 0, i, 0))
    den_spec = pl.BlockSpec((NC, Nb, 16), lambda i: (0, i, 0))
    return pl.pallas_call(
        functools.partial(_kx12_body, oc_prev, oc),
        grid=grid,
        in_specs=[num_spec, den_spec,
                  full((1, HEADS * oc_prev)), full((din, HEADS * oc)),
                  full((3, oc)), full((3, oc))],
        out_specs=[row(16)] * 5,
        out_shape=[jax.ShapeDtypeStruct((N, 16), jnp.float32)] * 5,
    )


def _tail_body(nsteps, n_total,
               num, den, bias2, meta, hn, cn,
               sp_w1, sp_b1, sp_w2, sp_b2, me_w, me_b, fc_w, fc_b,
               l0_wi, l0_wh, l0_bi, l0_bh, l1_wi, l1_wh, l1_bi, l1_bh,
               out_w, out_b, out, hs, cs, acc):
    i = pl.program_id(0)

    @pl.when(i == 0)
    def _():
        acc[...] = jnp.zeros_like(acc)

    parts = []
    num_b, den_b = num[...], den[...]
    for t in range(HEADS):
        numt = (num_b[t, 0] + num_b[t, 1])[:, :8]
        dent = (den_b[0, :, t] + den_b[1, :, t])[:, None]
        g = jnp.where(dent > 0.0, numt / dent, 0.0)
        parts.append(g.sum(axis=0, keepdims=True))
    acc[...] = acc[...] + jnp.concatenate(parts, axis=1)

    pooled = acc[...] + n_total * bias2[...]
    p = jnp.maximum(
        jnp.dot(pooled, sp_w1[...], preferred_element_type=jnp.float32)
        + sp_b1[...], 0.0)
    p = jnp.dot(p, sp_w2[...], preferred_element_type=jnp.float32) + sp_b2[...]
    m = jnp.maximum(
        jnp.dot(meta[...], me_w[...], preferred_element_type=jnp.float32)
        + me_b[...], 0.0)
    z = jnp.concatenate([p, m], axis=1)
    z = jnp.dot(z, fc_w[...], preferred_element_type=jnp.float32) + fc_b[...]
    z = jnp.where(z >= 0.0, z, 0.1 * z)

    step = z
    hs_new, cs_new = [], []
    for l, (wi, wh, bi, bh) in enumerate(
            ((l0_wi, l0_wh, l0_bi, l0_bh), (l1_wi, l1_wh, l1_bi, l1_bh))):
        g = (jnp.dot(step, wi[...], preferred_element_type=jnp.float32)
             + jnp.dot(hn[...][l:l + 1], wh[...],
                       preferred_element_type=jnp.float32)
             + bi[...] + bh[...])
        i_g, f_g, g_g, o_g = (g[:, 0:16], g[:, 16:32], g[:, 32:48], g[:, 48:64])
        c_new = (jax.nn.sigmoid(f_g) * cn[...][l:l + 1]
                 + jax.nn.sigmoid(i_g) * jnp.tanh(g_g))
        h_new = jax.nn.sigmoid(o_g) * jnp.tanh(c_new)
        hs_new.append(h_new)
        cs_new.append(c_new)
        step = h_new

    o = (jnp.dot(step, out_w[...], preferred_element_type=jnp.float32)
         + out_b[...])
    o = o - jnp.max(o, axis=1, keepdims=True)
    e = jnp.exp(o)
    out[...] = e / e.sum(axis=1, keepdims=True)
    hs[...] = jnp.concatenate(hs_new, axis=0)
    cs[...] = jnp.concatenate(cs_new, axis=0)


def _make_tail_kernel(N, Nb, n_real):
    grid = (N // Nb,)
    full = lambda shape: pl.BlockSpec(shape, lambda i: (0,) * len(shape))
    row = lambda w: pl.BlockSpec((Nb, w), lambda i: (i, 0))
    num_spec = pl.BlockSpec((3, NC, Nb, 16), lambda i: (0, 0, i, 0))
    den_spec = pl.BlockSpec((NC, Nb, 16), lambda i: (0, i, 0))
    return pl.pallas_call(
        functools.partial(_tail_body, N // Nb, n_real),
        grid=grid,
        in_specs=[num_spec, den_spec,
                  full((1, 24)), full((1, 6)), full((2, 16)), full((2, 16)),
                  full((24, 64)), full((1, 64)), full((64, 24)), full((1, 24)),
                  full((6, 8)), full((1, 8)), full((32, 16)), full((1, 16)),
                  full((16, 64)), full((16, 64)), full((1, 64)), full((1, 64)),
                  full((16, 64)), full((16, 64)), full((1, 64)), full((1, 64)),
                  full((16, 3)), full((1, 3))],
        out_specs=[full((1, 3)), full((2, 16)), full((2, 16))],
        out_shape=[jax.ShapeDtypeStruct((1, 3), jnp.float32),
                   jax.ShapeDtypeStruct((2, 16), jnp.float32),
                   jax.ShapeDtypeStruct((2, 16), jnp.float32)],
        scratch_shapes=[pltpu.VMEM((1, 24), jnp.float32)],
    )


def _make_edge_softmax_kernel(N, E, C):
    EperW = E // NW
    nch = EperW // C
    NpT = N // NS
    nvec = C * 4 // 16
    mesh = plsc.VectorSubcoreMesh(core_axis_name="c", subcore_axis_name="s")

    @functools.partial(
        pl.kernel,
        out_type=[jax.ShapeDtypeStruct((E, 4), jnp.float32),
                  jax.ShapeDtypeStruct((NC, N, 16), jnp.float32)],
        mesh=mesh,
        compiler_params=pltpu.CompilerParams(
            use_tc_tiling_on_sc=False, needs_layout_passes=False),
        scratch_types=[pltpu.VMEM((C,), jnp.int32),
                       pltpu.VMEM((C,), jnp.int32),
                       pltpu.VMEM((C, 16), jnp.float32),
                       pltpu.VMEM((C, 4), jnp.float32),
                       pltpu.VMEM((C, 4), jnp.float32),
                       pltpu.VMEM((C, 16), jnp.float32),
                       pltpu.VMEM_SHARED((N, 16), jnp.float32)],
    )
    def k(src_hbm, dst_hbm, als_hbm, ald_hbm, ale_hbm, z16_hbm,
          ex_hbm, den_hbm, vsrc, vdst, vacc16, vale, vex4,
          vex16, den_sh):
        cid = lax.axis_index("c")
        sid = lax.axis_index("s")
        wid = sid * NC + cid
        base = wid * EperW
        r0 = sid * NpT

        pltpu.sync_copy(z16_hbm.at[pl.ds(r0, NpT)], den_sh.at[pl.ds(r0, NpT)])
        pltpu.sync_copy(z16_hbm.at[pl.ds(0, C)], vex16)
        plsc.subcore_barrier()

        iota = lax.iota(jnp.int32, 16)
        riota = iota // 4
        ciota = iota % 4

        def chunk(kk, carry):
            e0 = base + kk * C
            pltpu.sync_copy(src_hbm.at[pl.ds(e0, C)], vsrc)
            pltpu.sync_copy(dst_hbm.at[pl.ds(e0, C)], vdst)
            pltpu.sync_copy(ale_hbm.at[pl.ds(e0, C)], vale)
            pltpu.sync_copy(als_hbm.at[vsrc], vacc16)
            pltpu.sync_copy(ald_hbm.at[vdst], vacc16, add=True)

            def vbody(i, c2):
                rows = i * 4 + riota
                v = (plsc.load_gather(vacc16, [rows, ciota])
                     + plsc.load_gather(vale, [rows, ciota]))
                v = jnp.where(v >= 0.0, v, 0.2 * v)
                v = jnp.exp(v)
                plsc.store_scatter(vex4, [rows, ciota], v)
                plsc.store_scatter(vex16, [rows, ciota], v)
                return c2

            lax.fori_loop(0, nvec, vbody, 0)
            pltpu.sync_copy(vex4, ex_hbm.at[pl.ds(e0, C)])
            pltpu.sync_copy(vex16, den_sh.at[vdst], add=True)
            return carry

        lax.fori_loop(0, nch, chunk, 0)
        plsc.subcore_barrier()
        pltpu.sync_copy(den_sh.at[pl.ds(r0, NpT)],
                        den_hbm.at[cid, pl.ds(r0, NpT)])

    return k


def _make_edge_agg_kernel(N, E, C):
    EperW = E // NW
    nch = EperW // C
    NpT = N // NS
    mesh = plsc.VectorSubcoreMesh(core_axis_name="c", subcore_axis_name="s")

    @functools.partial(
        pl.kernel,
        out_type=[jax.ShapeDtypeStruct((3, NC, N, 16), jnp.float32)],
        mesh=mesh,
        compiler_params=pltpu.CompilerParams(
            use_tc_tiling_on_sc=False, needs_layout_passes=False),
        scratch_types=[pltpu.VMEM((C,), jnp.int32),
                       pltpu.VMEM((C,), jnp.int32),
                       pltpu.VMEM((C, 4), jnp.float32),
                       pltpu.VMEM((C, 16), jnp.float32),
                       pltpu.VMEM_SHARED((N, 16), jnp.float32)],
    )
    def k(src_hbm, dst_hbm, ex_hbm, h0_hbm, h1_hbm, h2_hbm, z16_hbm,
          num_hbm, vsrc, vdst, vex, vh, num_sh):
        cid = lax.axis_index("c")
        sid = lax.axis_index("s")
        wid = sid * NC + cid
        base = wid * EperW
        r0 = sid * NpT

        for t in range(HEADS):
            h_hbm = (h0_hbm, h1_hbm, h2_hbm)[t]
            pltpu.sync_copy(z16_hbm.at[pl.ds(r0, NpT)],
                            num_sh.at[pl.ds(r0, NpT)])
            plsc.subcore_barrier()

            def chunk(kk, carry):
                e0 = base + kk * C
                pltpu.sync_copy(src_hbm.at[pl.ds(e0, C)], vsrc)
                pltpu.sync_copy(dst_hbm.at[pl.ds(e0, C)], vdst)
                pltpu.sync_copy(ex_hbm.at[pl.ds(e0, C)], vex)
                pltpu.sync_copy(h_hbm.at[vsrc], vh)

                tvec = jnp.full((16,), t, jnp.int32)

                def ebody(c2, cc):
                    s = plsc.load_gather(
                        vex, [jnp.full((16,), c2, jnp.int32), tvec])
                    vh[c2, :] = vh[c2, :] * s
                    return cc

                lax.fori_loop(0, C, ebody, 0)
                pltpu.sync_copy(vh, num_sh.at[vdst], add=True)
                return carry

            lax.fori_loop(0, nch, chunk, 0)
            plsc.subcore_barrier()
            pltpu.sync_copy(num_sh.at[pl.ds(r0, NpT)],
                            num_hbm.at[t, cid, pl.ds(r0, NpT)])
            plsc.subcore_barrier()

    return k


def kernel(x, edge_index, edge_attr, meta_data, hn, cn,
           W0, We0, asrc0, adst0, aedge0, bias0,
           W1, We1, asrc1, adst1, aedge1, bias1,
           W2, We2, asrc2, adst2, aedge2, bias2,
           sp_w1, sp_b1, sp_w2, sp_b2, me_w, me_b, fc_w, fc_b,
           l0_wi, l0_wh, l0_bi, l0_bh, l1_wi, l1_wh, l1_bi, l1_bh,
           out_w, out_b):
    N = x.shape[0]
    E = edge_index.shape[1]
    N2 = ((N + 2047) // 2048) * 2048
    C = 1000
    CA = 400
    Nb = 2048
    Eb = 4000

    src = edge_index[0]
    dst = edge_index[1]
    xp = jnp.pad(x, ((0, N2 - N), (0, 0)))
    z16 = jnp.zeros((N2, 16), jnp.float32)

    ale0, ale1, ale2 = _make_ale_kernel(E, Eb)(
        edge_attr, We0, aedge0, We1, aedge1, We2, aedge2)

    edge_softmax = _make_edge_softmax_kernel(N2, E, CA)
    edge_agg = _make_edge_agg_kernel(N2, E, C)

    h0, h1, h2, als, ald = _make_kx0_kernel(N2, Nb)(xp, W0, asrc0, adst0)
    ex, den = edge_softmax(src, dst, als, ald, ale0, z16)
    num, = edge_agg(src, dst, ex, h0, h1, h2, z16)

    h0, h1, h2, als, ald = _make_kx12_kernel(N2, Nb, 16, 16, 48)(
        num, den, bias0.reshape(1, 48), W1, asrc1, adst1)
    ex, den = edge_softmax(src, dst, als, ald, ale1, z16)
    num, = edge_agg(src, dst, ex, h0, h1, h2, z16)

    h0, h1, h2, als, ald = _make_kx12_kernel(N2, Nb, 16, 8, 48)(
        num, den, bias1.reshape(1, 48), W2, asrc2, adst2)
    ex, den = edge_softmax(src, dst, als, ald, ale2, z16)
    num, = edge_agg(src, dst, ex, h0, h1, h2, z16)

    out, hs, cs = _make_tail_kernel(N2, Nb, float(N))(
        num, den,
        bias2.reshape(1, 24), meta_data.reshape(1, 6), hn, cn,
        sp_w1, sp_b1.reshape(1, 64), sp_w2, sp_b2.reshape(1, 24),
        me_w, me_b.reshape(1, 8), fc_w, fc_b.reshape(1, 16),
        l0_wi, l0_wh, l0_bi.reshape(1, 64), l0_bh.reshape(1, 64),
        l1_wi, l1_wh, l1_bi.reshape(1, 64), l1_bh.reshape(1, 64),
        out_w, out_b.reshape(1, 3))

    return (out, hs, cs)


# speedup vs baseline: 52.2797x; 1.0172x over previous
import functools

import jax
import jax.numpy as jnp
from jax import lax
from jax.experimental import pallas as pl
from jax.experimental.pallas import tpu as pltpu
from jax.experimental.pallas import tpu_sc as plsc

HEADS = 3
NC = 2
NS = 16
NW = NC * NS


def _ale_body(ocs, ea, We0, ae0, We1, ae1, We2, ae2, o0, o1, o2):
    ea_blk = ea[...]
    eb = ea_blk.shape[0]
    for (We, ae, o, oc) in ((We0, ae0, o0, ocs[0]),
                            (We1, ae1, o1, ocs[1]),
                            (We2, ae2, o2, ocs[2])):
        cols = []
        for t in range(HEADS):
            m_t = (We[:, t * oc:(t + 1) * oc] * ae[t, :][None, :]).sum(
                axis=1, keepdims=True)
            cols.append(jnp.dot(ea_blk, m_t, preferred_element_type=jnp.float32))
        o[...] = jnp.concatenate(
            cols + [jnp.zeros((eb, 1), jnp.float32)], axis=1)


def _make_ale_kernel(E, Eb):
    grid = (E // Eb,)
    full = lambda shape: pl.BlockSpec(shape, lambda i: (0,) * len(shape))
    return pl.pallas_call(
        functools.partial(_ale_body, (16, 16, 8)),
        grid=grid,
        in_specs=[
            pl.BlockSpec((Eb, 2), lambda i: (i, 0)),
            full((2, 48)), full((3, 16)),
            full((2, 48)), full((3, 16)),
            full((2, 24)), full((3, 8)),
        ],
        out_specs=[pl.BlockSpec((Eb, 4), lambda i: (i, 0))] * 3,
        out_shape=[jax.ShapeDtypeStruct((E, 4), jnp.float32)] * 3,
    )


def _node_proj(h, a_s, a_d, oc, nb):
    hts, als_cols, ald_cols = [], [], []
    for t in range(HEADS):
        ht = h[:, t * oc:(t + 1) * oc]
        if oc < 16:
            hts.append(jnp.concatenate(
                [ht, jnp.zeros((nb, 16 - oc), jnp.float32)], axis=1))
        else:
            hts.append(ht)
        als_cols.append((ht * a_s[t, :][None, :]).sum(axis=1, keepdims=True))
        ald_cols.append((ht * a_d[t, :][None, :]).sum(axis=1, keepdims=True))
    z13 = jnp.zeros((nb, 13), jnp.float32)
    als = jnp.concatenate(als_cols + [z13], axis=1)
    ald = jnp.concatenate(ald_cols + [z13], axis=1)
    return hts, als, ald


def _kx0_body(x, W, a_s, a_d, h0, h1, h2, als, ald):
    xb = x[...]
    h = jnp.dot(xb, W[...], preferred_element_type=jnp.float32)
    hts, als_b, ald_b = _node_proj(h, a_s[...], a_d[...], 16, xb.shape[0])
    h0[...], h1[...], h2[...] = hts
    als[...], ald[...] = als_b, ald_b


def _make_kx0_kernel(N, Nb):
    grid = (N // Nb,)
    full = lambda shape: pl.BlockSpec(shape, lambda i: (0,) * len(shape))
    row = lambda w: pl.BlockSpec((Nb, w), lambda i: (i, 0))
    return pl.pallas_call(
        _kx0_body,
        grid=grid,
        in_specs=[row(3), full((3, 48)), full((3, 16)), full((3, 16))],
        out_specs=[row(16)] * 5,
        out_shape=[jax.ShapeDtypeStruct((N, 16), jnp.float32)] * 5,
    )


def _gat_epilogue(num, den, bias, oc_prev):
    outs = []
    for t in range(HEADS):
        numt = (num[t, 0] + num[t, 1])[:, :oc_prev]
        dent = (den[0, :, t] + den[1, :, t])[:, None]
        g = jnp.where(dent > 0.0, numt / dent, 0.0)
        g = g + bias[0, t * oc_prev:(t + 1) * oc_prev][None, :]
        outs.append(g)
    x = jnp.concatenate(outs, axis=1)
    return jnp.where(x > 0.0, x, jnp.exp(x) - 1.0)


def _kx12_body(oc_prev, oc, num, den, bias, W, a_s, a_d,
               h0, h1, h2, als, ald):
    x = _gat_epilogue(num[...], den[...], bias[...], oc_prev)
    h = jnp.dot(x, W[...], preferred_element_type=jnp.float32)
    hts, als_b, ald_b = _node_proj(h, a_s[...], a_d[...], oc, x.shape[0])
    h0[...], h1[...], h2[...] = hts
    als[...], ald[...] = als_b, ald_b


def _make_kx12_kernel(N, Nb, oc_prev, oc, din):
    grid = (N // Nb,)
    full = lambda shape: pl.BlockSpec(shape, lambda i: (0,) * len(shape))
    row = lambda w: pl.BlockSpec((Nb, w), lambda i: (i, 0))
    num_spec = pl.BlockSpec((3, NC, Nb, 16), lambda i: (0, 0, i, 0))
    den_spec = pl.BlockSpec((NC, Nb, 16), lambda i: (0, i, 0))
    return pl.pallas_call(
        functools.partial(_kx12_body, oc_prev, oc),
        grid=grid,
        in_specs=[num_spec, den_spec,
                  full((1, HEADS * oc_prev)), full((din, HEADS * oc)),
                  full((3, oc)), full((3, oc))],
        out_specs=[row(16)] * 5,
        out_shape=[jax.ShapeDtypeStruct((N, 16), jnp.float32)] * 5,
    )


def _tail_body(nsteps, n_total,
               num, den, bias2, meta, hn, cn,
               sp_w1, sp_b1, sp_w2, sp_b2, me_w, me_b, fc_w, fc_b,
               l0_wi, l0_wh, l0_bi, l0_bh, l1_wi, l1_wh, l1_bi, l1_bh,
               out_w, out_b, out, hs, cs, acc):
    i = pl.program_id(0)

    @pl.when(i == 0)
    def _():
        acc[...] = jnp.zeros_like(acc)

    parts = []
    num_b, den_b = num[...], den[...]
    for t in range(HEADS):
        numt = (num_b[t, 0] + num_b[t, 1])[:, :8]
        dent = (den_b[0, :, t] + den_b[1, :, t])[:, None]
        g = jnp.where(dent > 0.0, numt / dent, 0.0)
        parts.append(g.sum(axis=0, keepdims=True))
    acc[...] = acc[...] + jnp.concatenate(parts, axis=1)

    pooled = acc[...] + n_total * bias2[...]
    p = jnp.maximum(
        jnp.dot(pooled, sp_w1[...], preferred_element_type=jnp.float32)
        + sp_b1[...], 0.0)
    p = jnp.dot(p, sp_w2[...], preferred_element_type=jnp.float32) + sp_b2[...]
    m = jnp.maximum(
        jnp.dot(meta[...], me_w[...], preferred_element_type=jnp.float32)
        + me_b[...], 0.0)
    z = jnp.concatenate([p, m], axis=1)
    z = jnp.dot(z, fc_w[...], preferred_element_type=jnp.float32) + fc_b[...]
    z = jnp.where(z >= 0.0, z, 0.1 * z)

    step = z
    hs_new, cs_new = [], []
    for l, (wi, wh, bi, bh) in enumerate(
            ((l0_wi, l0_wh, l0_bi, l0_bh), (l1_wi, l1_wh, l1_bi, l1_bh))):
        g = (jnp.dot(step, wi[...], preferred_element_type=jnp.float32)
             + jnp.dot(hn[...][l:l + 1], wh[...],
                       preferred_element_type=jnp.float32)
             + bi[...] + bh[...])
        i_g, f_g, g_g, o_g = (g[:, 0:16], g[:, 16:32], g[:, 32:48], g[:, 48:64])
        c_new = (jax.nn.sigmoid(f_g) * cn[...][l:l + 1]
                 + jax.nn.sigmoid(i_g) * jnp.tanh(g_g))
        h_new = jax.nn.sigmoid(o_g) * jnp.tanh(c_new)
        hs_new.append(h_new)
        cs_new.append(c_new)
        step = h_new

    o = (jnp.dot(step, out_w[...], preferred_element_type=jnp.float32)
         + out_b[...])
    o = o - jnp.max(o, axis=1, keepdims=True)
    e = jnp.exp(o)
    out[...] = e / e.sum(axis=1, keepdims=True)
    hs[...] = jnp.concatenate(hs_new, axis=0)
    cs[...] = jnp.concatenate(cs_new, axis=0)


def _make_tail_kernel(N, Nb, n_real):
    grid = (N // Nb,)
    full = lambda shape: pl.BlockSpec(shape, lambda i: (0,) * len(shape))
    row = lambda w: pl.BlockSpec((Nb, w), lambda i: (i, 0))
    num_spec = pl.BlockSpec((3, NC, Nb, 16), lambda i: (0, 0, i, 0))
    den_spec = pl.BlockSpec((NC, Nb, 16), lambda i: (0, i, 0))
    return pl.pallas_call(
        functools.partial(_tail_body, N // Nb, n_real),
        grid=grid,
        in_specs=[num_spec, den_spec,
                  full((1, 24)), full((1, 6)), full((2, 16)), full((2, 16)),
                  full((24, 64)), full((1, 64)), full((64, 24)), full((1, 24)),
                  full((6, 8)), full((1, 8)), full((32, 16)), full((1, 16)),
                  full((16, 64)), full((16, 64)), full((1, 64)), full((1, 64)),
                  full((16, 64)), full((16, 64)), full((1, 64)), full((1, 64)),
                  full((16, 3)), full((1, 3))],
        out_specs=[full((1, 3)), full((2, 16)), full((2, 16))],
        out_shape=[jax.ShapeDtypeStruct((1, 3), jnp.float32),
                   jax.ShapeDtypeStruct((2, 16), jnp.float32),
                   jax.ShapeDtypeStruct((2, 16), jnp.float32)],
        scratch_shapes=[pltpu.VMEM((1, 24), jnp.float32)],
    )


def _make_edge_softmax_kernel(N, E, C):
    EperW = E // NW
    nch = EperW // C
    NpT = N // NS
    nvec = C * 4 // 16
    mesh = plsc.VectorSubcoreMesh(core_axis_name="c", subcore_axis_name="s")

    @functools.partial(
        pl.kernel,
        out_type=[jax.ShapeDtypeStruct((E, 4), jnp.float32),
                  jax.ShapeDtypeStruct((NC, N, 16), jnp.float32)],
        mesh=mesh,
        compiler_params=pltpu.CompilerParams(
            use_tc_tiling_on_sc=False, needs_layout_passes=False),
        scratch_types=[pltpu.VMEM((C,), jnp.int32),
                       pltpu.VMEM((C,), jnp.int32),
                       pltpu.VMEM((C, 16), jnp.float32),
                       pltpu.VMEM((C, 4), jnp.float32),
                       pltpu.VMEM((C, 4), jnp.float32),
                       pltpu.VMEM((C, 16), jnp.float32),
                       pltpu.VMEM_SHARED((N, 16), jnp.float32)],
    )
    def k(src_hbm, dst_hbm, als_hbm, ald_hbm, ale_hbm, z16_hbm,
          ex_hbm, den_hbm, vsrc, vdst, vacc16, vale, vex4,
          vex16, den_sh):
        cid = lax.axis_index("c")
        sid = lax.axis_index("s")
        wid = sid * NC + cid
        base = wid * EperW
        r0 = sid * NpT

        pltpu.sync_copy(z16_hbm.at[pl.ds(r0, NpT)], den_sh.at[pl.ds(r0, NpT)])
        pltpu.sync_copy(z16_hbm.at[pl.ds(0, C)], vex16)
        plsc.subcore_barrier()

        iota = lax.iota(jnp.int32, 16)
        riota = iota // 4
        ciota = iota % 4

        def chunk(kk, carry):
            e0 = base + kk * C
            pltpu.sync_copy(src_hbm.at[pl.ds(e0, C)], vsrc)
            pltpu.sync_copy(dst_hbm.at[pl.ds(e0, C)], vdst)
            pltpu.sync_copy(ale_hbm.at[pl.ds(e0, C)], vale)
            pltpu.sync_copy(als_hbm.at[vsrc], vacc16)
            pltpu.sync_copy(ald_hbm.at[vdst], vacc16, add=True)

            def vbody(i, c2):
                for u in range(4):
                    rows = (i * 4 + u) * 4 + riota
                    v = (plsc.load_gather(vacc16, [rows, ciota])
                         + plsc.load_gather(vale, [rows, ciota]))
                    v = jnp.where(v >= 0.0, v, 0.2 * v)
                    v = jnp.exp(v)
                    plsc.store_scatter(vex4, [rows, ciota], v)
                    plsc.store_scatter(vex16, [rows, ciota], v)
                return c2

            lax.fori_loop(0, nvec // 4, vbody, 0)
            pltpu.sync_copy(vex4, ex_hbm.at[pl.ds(e0, C)])
            pltpu.sync_copy(vex16, den_sh.at[vdst], add=True)
            return carry

        lax.fori_loop(0, nch, chunk, 0)
        plsc.subcore_barrier()
        pltpu.sync_copy(den_sh.at[pl.ds(r0, NpT)],
                        den_hbm.at[cid, pl.ds(r0, NpT)])

    return k


def _make_edge_agg_kernel(N, E, C):
    EperW = E // NW
    nch = EperW // C
    NpT = N // NS
    mesh = plsc.VectorSubcoreMesh(core_axis_name="c", subcore_axis_name="s")

    @functools.partial(
        pl.kernel,
        out_type=[jax.ShapeDtypeStruct((3, NC, N, 16), jnp.float32)],
        mesh=mesh,
        compiler_params=pltpu.CompilerParams(
            use_tc_tiling_on_sc=False, needs_layout_passes=False),
        scratch_types=[pltpu.VMEM((C,), jnp.int32),
                       pltpu.VMEM((C,), jnp.int32),
                       pltpu.VMEM((C, 4), jnp.float32),
                       pltpu.VMEM((C, 16), jnp.float32),
                       pltpu.VMEM_SHARED((N, 16), jnp.float32)],
    )
    def k(src_hbm, dst_hbm, ex_hbm, h0_hbm, h1_hbm, h2_hbm, z16_hbm,
          num_hbm, vsrc, vdst, vex, vh, num_sh):
        cid = lax.axis_index("c")
        sid = lax.axis_index("s")
        wid = sid * NC + cid
        base = wid * EperW
        r0 = sid * NpT

        for t in range(HEADS):
            h_hbm = (h0_hbm, h1_hbm, h2_hbm)[t]
            pltpu.sync_copy(z16_hbm.at[pl.ds(r0, NpT)],
                            num_sh.at[pl.ds(r0, NpT)])
            plsc.subcore_barrier()

            def chunk(kk, carry):
                e0 = base + kk * C
                pltpu.sync_copy(src_hbm.at[pl.ds(e0, C)], vsrc)
                pltpu.sync_copy(dst_hbm.at[pl.ds(e0, C)], vdst)
                pltpu.sync_copy(ex_hbm.at[pl.ds(e0, C)], vex)
                pltpu.sync_copy(h_hbm.at[vsrc], vh)

                tvec = jnp.full((16,), t, jnp.int32)

                def ebody(c2, cc):
                    for u in range(8):
                        c3 = c2 * 8 + u
                        s = plsc.load_gather(
                            vex, [jnp.full((16,), c3, jnp.int32), tvec])
                        vh[c3, :] = vh[c3, :] * s
                    return cc

                lax.fori_loop(0, C // 8, ebody, 0)
                pltpu.sync_copy(vh, num_sh.at[vdst], add=True)
                return carry

            lax.fori_loop(0, nch, chunk, 0)
            plsc.subcore_barrier()
            pltpu.sync_copy(num_sh.at[pl.ds(r0, NpT)],
                            num_hbm.at[t, cid, pl.ds(r0, NpT)])
            plsc.subcore_barrier()

    return k


def kernel(x, edge_index, edge_attr, meta_data, hn, cn,
           W0, We0, asrc0, adst0, aedge0, bias0,
           W1, We1, asrc1, adst1, aedge1, bias1,
           W2, We2, asrc2, adst2, aedge2, bias2,
           sp_w1, sp_b1, sp_w2, sp_b2, me_w, me_b, fc_w, fc_b,
           l0_wi, l0_wh, l0_bi, l0_bh, l1_wi, l1_wh, l1_bi, l1_bh,
           out_w, out_b):
    N = x.shape[0]
    E = edge_index.shape[1]
    N2 = ((N + 2047) // 2048) * 2048
    C = 1000
    CA = 400
    Nb = 2048
    Eb = 4000

    src = edge_index[0]
    dst = edge_index[1]
    xp = jnp.pad(x, ((0, N2 - N), (0, 0)))
    z16 = jnp.zeros((N2, 16), jnp.float32)

    ale0, ale1, ale2 = _make_ale_kernel(E, Eb)(
        edge_attr, We0, aedge0, We1, aedge1, We2, aedge2)

    edge_softmax = _make_edge_softmax_kernel(N2, E, CA)
    edge_agg = _make_edge_agg_kernel(N2, E, C)

    h0, h1, h2, als, ald = _make_kx0_kernel(N2, Nb)(xp, W0, asrc0, adst0)
    ex, den = edge_softmax(src, dst, als, ald, ale0, z16)
    num, = edge_agg(src, dst, ex, h0, h1, h2, z16)

    h0, h1, h2, als, ald = _make_kx12_kernel(N2, Nb, 16, 16, 48)(
        num, den, bias0.reshape(1, 48), W1, asrc1, adst1)
    ex, den = edge_softmax(src, dst, als, ald, ale1, z16)
    num, = edge_agg(src, dst, ex, h0, h1, h2, z16)

    h0, h1, h2, als, ald = _make_kx12_kernel(N2, Nb, 16, 8, 48)(
        num, den, bias1.reshape(1, 48), W2, asrc2, adst2)
    ex, den = edge_softmax(src, dst, als, ald, ale2, z16)
    num, = edge_agg(src, dst, ex, h0, h1, h2, z16)

    out, hs, cs = _make_tail_kernel(N2, Nb, float(N))(
        num, den,
        bias2.reshape(1, 24), meta_data.reshape(1, 6), hn, cn,
        sp_w1, sp_b1.reshape(1, 64), sp_w2, sp_b2.reshape(1, 24),
        me_w, me_b.reshape(1, 8), fc_w, fc_b.reshape(1, 16),
        l0_wi, l0_wh, l0_bi.reshape(1, 64), l0_bh.reshape(1, 64),
        l1_wi, l1_wh, l1_bi.reshape(1, 64), l1_bh.reshape(1, 64),
        out_w, out_b.reshape(1, 3))

    return (out, hs, cs)


# speedup vs baseline: 55.7705x; 1.0668x over previous
import functools

import jax
import jax.numpy as jnp
from jax import lax
from jax.experimental import pallas as pl
from jax.experimental.pallas import tpu as pltpu
from jax.experimental.pallas import tpu_sc as plsc

HEADS = 3
NC = 2
NS = 16
NW = NC * NS


def _ale_body(ocs, ea, We0, ae0, We1, ae1, We2, ae2, o0, o1, o2):
    ea_blk = ea[...]
    eb = ea_blk.shape[0]
    for (We, ae, o, oc) in ((We0, ae0, o0, ocs[0]),
                            (We1, ae1, o1, ocs[1]),
                            (We2, ae2, o2, ocs[2])):
        cols = []
        for t in range(HEADS):
            m_t = (We[:, t * oc:(t + 1) * oc] * ae[t, :][None, :]).sum(
                axis=1, keepdims=True)
            cols.append(jnp.dot(ea_blk, m_t, preferred_element_type=jnp.float32))
        o[...] = jnp.concatenate(
            cols + [jnp.zeros((eb, 1), jnp.float32)], axis=1)


def _make_ale_kernel(E, Eb):
    grid = (E // Eb,)
    full = lambda shape: pl.BlockSpec(shape, lambda i: (0,) * len(shape))
    return pl.pallas_call(
        functools.partial(_ale_body, (16, 16, 8)),
        grid=grid,
        in_specs=[
            pl.BlockSpec((Eb, 2), lambda i: (i, 0)),
            full((2, 48)), full((3, 16)),
            full((2, 48)), full((3, 16)),
            full((2, 24)), full((3, 8)),
        ],
        out_specs=[pl.BlockSpec((Eb, 4), lambda i: (i, 0))] * 3,
        out_shape=[jax.ShapeDtypeStruct((E, 4), jnp.float32)] * 3,
    )


def _node_proj(h, a_s, a_d, oc, nb):
    hts, als_cols, ald_cols = [], [], []
    for t in range(HEADS):
        ht = h[:, t * oc:(t + 1) * oc]
        if oc < 16:
            hts.append(jnp.concatenate(
                [ht, jnp.zeros((nb, 16 - oc), jnp.float32)], axis=1))
        else:
            hts.append(ht)
        als_cols.append((ht * a_s[t, :][None, :]).sum(axis=1, keepdims=True))
        ald_cols.append((ht * a_d[t, :][None, :]).sum(axis=1, keepdims=True))
    z13 = jnp.zeros((nb, 13), jnp.float32)
    als = jnp.concatenate(als_cols + [z13], axis=1)
    ald = jnp.concatenate(ald_cols + [z13], axis=1)
    return hts, als, ald


def _kx0_body(x, W, a_s, a_d, h0, h1, h2, als, ald):
    xb = x[...]
    h = jnp.dot(xb, W[...], preferred_element_type=jnp.float32)
    hts, als_b, ald_b = _node_proj(h, a_s[...], a_d[...], 16, xb.shape[0])
    h0[...], h1[...], h2[...] = hts
    als[...], ald[...] = als_b, ald_b


def _make_kx0_kernel(N, Nb):
    grid = (N // Nb,)
    full = lambda shape: pl.BlockSpec(shape, lambda i: (0,) * len(shape))
    row = lambda w: pl.BlockSpec((Nb, w), lambda i: (i, 0))
    return pl.pallas_call(
        _kx0_body,
        grid=grid,
        in_specs=[row(3), full((3, 48)), full((3, 16)), full((3, 16))],
        out_specs=[row(16)] * 5,
        out_shape=[jax.ShapeDtypeStruct((N, 16), jnp.float32)] * 5,
    )


def _gat_epilogue(num, den, bias, oc_prev):
    outs = []
    for t in range(HEADS):
        numt = (num[t, 0] + num[t, 1])[:, :oc_prev]
        dent = (den[0, :, t] + den[1, :, t])[:, None]
        g = jnp.where(dent > 0.0, numt / dent, 0.0)
        g = g + bias[0, t * oc_prev:(t + 1) * oc_prev][None, :]
        outs.append(g)
    x = jnp.concatenate(outs, axis=1)
    return jnp.where(x > 0.0, x, jnp.exp(x) - 1.0)


def _kx12_body(oc_prev, oc, num, den, bias, W, a_s, a_d,
               h0, h1, h2, als, ald):
    x = _gat_epilogue(num[...], den[...], bias[...], oc_prev)
    h = jnp.dot(x, W[...], preferred_element_type=jnp.float32)
    hts, als_b, ald_b = _node_proj(h, a_s[...], a_d[...], oc, x.shape[0])
    h0[...], h1[...], h2[...] = hts
    als[...], ald[...] = als_b, ald_b


def _make_kx12_kernel(N, Nb, oc_prev, oc, din):
    grid = (N // Nb,)
    full = lambda shape: pl.BlockSpec(shape, lambda i: (0,) * len(shape))
    row = lambda w: pl.BlockSpec((Nb, w), lambda i: (i, 0))
    num_spec = pl.BlockSpec((3, NC, Nb, 16), lambda i: (0, 0, i, 0))
    den_spec = pl.BlockSpec((NC, Nb, 16), lambda i: (0, i, 0))
    return pl.pallas_call(
        functools.partial(_kx12_body, oc_prev, oc),
        grid=grid,
        in_specs=[num_spec, den_spec,
                  full((1, HEADS * oc_prev)), full((din, HEADS * oc)),
                  full((3, oc)), full((3, oc))],
        out_specs=[row(16)] * 5,
        out_shape=[jax.ShapeDtypeStruct((N, 16), jnp.float32)] * 5,
    )


def _tail_body(nsteps, n_total,
               num, den, bias2, meta, hn, cn,
               sp_w1, sp_b1, sp_w2, sp_b2, me_w, me_b, fc_w, fc_b,
               l0_wi, l0_wh, l0_bi, l0_bh, l1_wi, l1_wh, l1_bi, l1_bh,
               out_w, out_b, out, hs, cs, acc):
    i = pl.program_id(0)

    @pl.when(i == 0)
    def _():
        acc[...] = jnp.zeros_like(acc)

    parts = []
    num_b, den_b = num[...], den[...]
    for t in range(HEADS):
        numt = (num_b[t, 0] + num_b[t, 1])[:, :8]
        dent = (den_b[0, :, t] + den_b[1, :, t])[:, None]
        g = jnp.where(dent > 0.0, numt / dent, 0.0)
        parts.append(g.sum(axis=0, keepdims=True))
    acc[...] = acc[...] + jnp.concatenate(parts, axis=1)

    pooled = acc[...] + n_total * bias2[...]
    p = jnp.maximum(
        jnp.dot(pooled, sp_w1[...], preferred_element_type=jnp.float32)
        + sp_b1[...], 0.0)
    p = jnp.dot(p, sp_w2[...], preferred_element_type=jnp.float32) + sp_b2[...]
    m = jnp.maximum(
        jnp.dot(meta[...], me_w[...], preferred_element_type=jnp.float32)
        + me_b[...], 0.0)
    z = jnp.concatenate([p, m], axis=1)
    z = jnp.dot(z, fc_w[...], preferred_element_type=jnp.float32) + fc_b[...]
    z = jnp.where(z >= 0.0, z, 0.1 * z)

    step = z
    hs_new, cs_new = [], []
    for l, (wi, wh, bi, bh) in enumerate(
            ((l0_wi, l0_wh, l0_bi, l0_bh), (l1_wi, l1_wh, l1_bi, l1_bh))):
        g = (jnp.dot(step, wi[...], preferred_element_type=jnp.float32)
             + jnp.dot(hn[...][l:l + 1], wh[...],
                       preferred_element_type=jnp.float32)
             + bi[...] + bh[...])
        i_g, f_g, g_g, o_g = (g[:, 0:16], g[:, 16:32], g[:, 32:48], g[:, 48:64])
        c_new = (jax.nn.sigmoid(f_g) * cn[...][l:l + 1]
                 + jax.nn.sigmoid(i_g) * jnp.tanh(g_g))
        h_new = jax.nn.sigmoid(o_g) * jnp.tanh(c_new)
        hs_new.append(h_new)
        cs_new.append(c_new)
        step = h_new

    o = (jnp.dot(step, out_w[...], preferred_element_type=jnp.float32)
         + out_b[...])
    o = o - jnp.max(o, axis=1, keepdims=True)
    e = jnp.exp(o)
    out[...] = e / e.sum(axis=1, keepdims=True)
    hs[...] = jnp.concatenate(hs_new, axis=0)
    cs[...] = jnp.concatenate(cs_new, axis=0)


def _make_tail_kernel(N, Nb, n_real):
    grid = (N // Nb,)
    full = lambda shape: pl.BlockSpec(shape, lambda i: (0,) * len(shape))
    row = lambda w: pl.BlockSpec((Nb, w), lambda i: (i, 0))
    num_spec = pl.BlockSpec((3, NC, Nb, 16), lambda i: (0, 0, i, 0))
    den_spec = pl.BlockSpec((NC, Nb, 16), lambda i: (0, i, 0))
    return pl.pallas_call(
        functools.partial(_tail_body, N // Nb, n_real),
        grid=grid,
        in_specs=[num_spec, den_spec,
                  full((1, 24)), full((1, 6)), full((2, 16)), full((2, 16)),
                  full((24, 64)), full((1, 64)), full((64, 24)), full((1, 24)),
                  full((6, 8)), full((1, 8)), full((32, 16)), full((1, 16)),
                  full((16, 64)), full((16, 64)), full((1, 64)), full((1, 64)),
                  full((16, 64)), full((16, 64)), full((1, 64)), full((1, 64)),
                  full((16, 3)), full((1, 3))],
        out_specs=[full((1, 3)), full((2, 16)), full((2, 16))],
        out_shape=[jax.ShapeDtypeStruct((1, 3), jnp.float32),
                   jax.ShapeDtypeStruct((2, 16), jnp.float32),
                   jax.ShapeDtypeStruct((2, 16), jnp.float32)],
        scratch_shapes=[pltpu.VMEM((1, 24), jnp.float32)],
    )


def _make_edge_softmax_kernel(N, E, C):
    EperW = E // NW
    nch = EperW // C
    NpT = N // NS
    nvec = C * 4 // 16
    mesh = plsc.VectorSubcoreMesh(core_axis_name="c", subcore_axis_name="s")

    @functools.partial(
        pl.kernel,
        out_type=[jax.ShapeDtypeStruct((E, 4), jnp.float32),
                  jax.ShapeDtypeStruct((NC, N, 16), jnp.float32)],
        mesh=mesh,
        compiler_params=pltpu.CompilerParams(
            use_tc_tiling_on_sc=False, needs_layout_passes=False),
        scratch_types=[pltpu.VMEM((C,), jnp.int32),
                       pltpu.VMEM((C,), jnp.int32),
                       pltpu.VMEM((C, 16), jnp.float32),
                       pltpu.VMEM((C, 4), jnp.float32),
                       pltpu.VMEM((C, 4), jnp.float32),
                       pltpu.VMEM((C, 16), jnp.float32),
                       pltpu.VMEM_SHARED((N, 16), jnp.float32)],
    )
    def k(src_hbm, dst_hbm, als_hbm, ald_hbm, ale_hbm, z16_hbm,
          ex_hbm, den_hbm, vsrc, vdst, vacc16, vale, vex4,
          vex16, den_sh):
        cid = lax.axis_index("c")
        sid = lax.axis_index("s")
        wid = sid * NC + cid
        base = wid * EperW
        r0 = sid * NpT

        pltpu.sync_copy(z16_hbm.at[pl.ds(r0, NpT)], den_sh.at[pl.ds(r0, NpT)])
        pltpu.sync_copy(z16_hbm.at[pl.ds(0, C)], vex16)
        plsc.subcore_barrier()

        iota = lax.iota(jnp.int32, 16)
        riota = iota // 4
        ciota = iota % 4

        def chunk(kk, carry):
            e0 = base + kk * C
            pltpu.sync_copy(src_hbm.at[pl.ds(e0, C)], vsrc)
            pltpu.sync_copy(dst_hbm.at[pl.ds(e0, C)], vdst)
            pltpu.sync_copy(ale_hbm.at[pl.ds(e0, C)], vale)
            pltpu.sync_copy(als_hbm.at[vsrc], vacc16)
            pltpu.sync_copy(ald_hbm.at[vdst], vacc16, add=True)

            def vbody(i, c2):
                for u in range(4):
                    rows = (i * 4 + u) * 4 + riota
                    v = (plsc.load_gather(vacc16, [rows, ciota])
                         + plsc.load_gather(vale, [rows, ciota]))
                    v = jnp.where(v >= 0.0, v, 0.2 * v)
                    v = jnp.exp(v)
                    plsc.store_scatter(vex4, [rows, ciota], v)
                    plsc.store_scatter(vex16, [rows, ciota], v)
                return c2

            lax.fori_loop(0, nvec // 4, vbody, 0)
            pltpu.sync_copy(vex4, ex_hbm.at[pl.ds(e0, C)])
            pltpu.sync_copy(vex16, den_sh.at[vdst], add=True)
            return carry

        lax.fori_loop(0, nch, chunk, 0)
        plsc.subcore_barrier()
        pltpu.sync_copy(den_sh.at[pl.ds(r0, NpT)],
                        den_hbm.at[cid, pl.ds(r0, NpT)])

    return k


def _make_edge_agg_kernel(N, E, C):
    EperW = E // NW
    nch = EperW // C
    assert nch % 2 == 1
    NpT = N // NS
    mesh = plsc.VectorSubcoreMesh(core_axis_name="c", subcore_axis_name="s")

    @functools.partial(
        pl.kernel,
        out_type=[jax.ShapeDtypeStruct((3, NC, N, 16), jnp.float32)],
        mesh=mesh,
        compiler_params=pltpu.CompilerParams(
            use_tc_tiling_on_sc=False, needs_layout_passes=False),
        scratch_types=[pltpu.VMEM((C,), jnp.int32),
                       pltpu.VMEM((C,), jnp.int32),
                       pltpu.VMEM((C, 4), jnp.float32),
                       pltpu.VMEM((C, 16), jnp.float32),
                       pltpu.VMEM((C,), jnp.int32),
                       pltpu.VMEM((C,), jnp.int32),
                       pltpu.VMEM((C, 4), jnp.float32),
                       pltpu.VMEM((C, 16), jnp.float32),
                       pltpu.SemaphoreType.DMA,
                       pltpu.SemaphoreType.DMA,
                       pltpu.SemaphoreType.DMA,
                       pltpu.SemaphoreType.DMA,
                       pltpu.VMEM_SHARED((N, 16), jnp.float32)],
    )
    def k(src_hbm, dst_hbm, ex_hbm, h0_hbm, h1_hbm, h2_hbm, z16_hbm,
          num_hbm, vsrc0, vdst0, vex0, vh0, vsrc1, vdst1, vex1, vh1,
          seml0, semg0, seml1, semg1, num_sh):
        cid = lax.axis_index("c")
        sid = lax.axis_index("s")
        wid = sid * NC + cid
        base = wid * EperW
        r0 = sid * NpT
        bufs = ((vsrc0, vdst0, vex0, vh0, seml0, semg0),
                (vsrc1, vdst1, vex1, vh1, seml1, semg1))

        def load_linear(e0, buf, sync):
            vsrc, vdst, vex, vh, seml, semg = buf
            if sync:
                pltpu.sync_copy(src_hbm.at[pl.ds(e0, C)], vsrc)
                pltpu.sync_copy(dst_hbm.at[pl.ds(e0, C)], vdst)
                pltpu.sync_copy(ex_hbm.at[pl.ds(e0, C)], vex)
            else:
                pltpu.async_copy(src_hbm.at[pl.ds(e0, C)], vsrc, seml)
                pltpu.async_copy(dst_hbm.at[pl.ds(e0, C)], vdst, seml)
                pltpu.async_copy(ex_hbm.at[pl.ds(e0, C)], vex, seml)

        def wait_linear(e0, buf):
            vsrc, vdst, vex, vh, seml, semg = buf
            pltpu.make_async_copy(src_hbm.at[pl.ds(e0, C)], vsrc, seml).wait()
            pltpu.make_async_copy(dst_hbm.at[pl.ds(e0, C)], vdst, seml).wait()
            pltpu.make_async_copy(ex_hbm.at[pl.ds(e0, C)], vex, seml).wait()

        for t in range(HEADS):
            h_hbm = (h0_hbm, h1_hbm, h2_hbm)[t]
            pltpu.sync_copy(z16_hbm.at[pl.ds(r0, NpT)],
                            num_sh.at[pl.ds(r0, NpT)])
            plsc.subcore_barrier()

            tvec = jnp.full((16,), t, jnp.int32)

            def compute(buf):
                vsrc, vdst, vex, vh, seml, semg = buf

                def ebody(c2, cc):
                    for u in range(8):
                        c3 = c2 * 8 + u
                        sc = plsc.load_gather(
                            vex, [jnp.full((16,), c3, jnp.int32), tvec])
                        vh[c3, :] = vh[c3, :] * sc
                    return cc

                lax.fori_loop(0, C // 8, ebody, 0)

            load_linear(base, bufs[0], True)
            pltpu.async_copy(h_hbm.at[vsrc0], vh0, semg0)

            def pair(kk2, carry):
                for par in range(2):
                    kk = kk2 * 2 + par
                    cur = bufs[par]
                    nxt = bufs[1 - par]
                    e0n = base + (kk + 1) * C
                    load_linear(e0n, nxt, False)
                    pltpu.make_async_copy(
                        h_hbm.at[cur[0]], cur[3], cur[5]).wait()
                    compute(cur)
                    wait_linear(e0n, nxt)
                    pltpu.async_copy(h_hbm.at[nxt[0]], nxt[3], nxt[5])
                    pltpu.sync_copy(cur[3], num_sh.at[cur[1]], add=True)
                return carry

            lax.fori_loop(0, (nch - 1) // 2, pair, 0)

            cur = bufs[0]
            pltpu.make_async_copy(h_hbm.at[cur[0]], cur[3], cur[5]).wait()
            compute(cur)
            pltpu.sync_copy(cur[3], num_sh.at[cur[1]], add=True)

            plsc.subcore_barrier()
            pltpu.sync_copy(num_sh.at[pl.ds(r0, NpT)],
                            num_hbm.at[t, cid, pl.ds(r0, NpT)])
            plsc.subcore_barrier()

    return k


def kernel(x, edge_index, edge_attr, meta_data, hn, cn,
           W0, We0, asrc0, adst0, aedge0, bias0,
           W1, We1, asrc1, adst1, aedge1, bias1,
           W2, We2, asrc2, adst2, aedge2, bias2,
           sp_w1, sp_b1, sp_w2, sp_b2, me_w, me_b, fc_w, fc_b,
           l0_wi, l0_wh, l0_bi, l0_bh, l1_wi, l1_wh, l1_bi, l1_bh,
           out_w, out_b):
    N = x.shape[0]
    E = edge_index.shape[1]
    N2 = ((N + 2047) // 2048) * 2048
    CA = 400
    CE = 400
    Nb = 2048
    Eb = 4000

    src = edge_index[0]
    dst = edge_index[1]
    xp = jnp.pad(x, ((0, N2 - N), (0, 0)))
    z16 = jnp.zeros((N2, 16), jnp.float32)

    ale0, ale1, ale2 = _make_ale_kernel(E, Eb)(
        edge_attr, We0, aedge0, We1, aedge1, We2, aedge2)

    edge_softmax = _make_edge_softmax_kernel(N2, E, CA)
    edge_agg = _make_edge_agg_kernel(N2, E, CE)

    h0, h1, h2, als, ald = _make_kx0_kernel(N2, Nb)(xp, W0, asrc0, adst0)
    ex, den = edge_softmax(src, dst, als, ald, ale0, z16)
    num, = edge_agg(src, dst, ex, h0, h1, h2, z16)

    h0, h1, h2, als, ald = _make_kx12_kernel(N2, Nb, 16, 16, 48)(
        num, den, bias0.reshape(1, 48), W1, asrc1, adst1)
    ex, den = edge_softmax(src, dst, als, ald, ale1, z16)
    num, = edge_agg(src, dst, ex, h0, h1, h2, z16)

    h0, h1, h2, als, ald = _make_kx12_kernel(N2, Nb, 16, 8, 48)(
        num, den, bias1.reshape(1, 48), W2, asrc2, adst2)
    ex, den = edge_softmax(src, dst, als, ald, ale2, z16)
    num, = edge_agg(src, dst, ex, h0, h1, h2, z16)

    out, hs, cs = _make_tail_kernel(N2, Nb, float(N))(
        num, den,
        bias2.reshape(1, 24), meta_data.reshape(1, 6), hn, cn,
        sp_w1, sp_b1.reshape(1, 64), sp_w2, sp_b2.reshape(1, 24),
        me_w, me_b.reshape(1, 8), fc_w, fc_b.reshape(1, 16),
        l0_wi, l0_wh, l0_bi.reshape(1, 64), l0_bh.reshape(1, 64),
        l1_wi, l1_wh, l1_bi.reshape(1, 64), l1_bh.reshape(1, 64),
        out_w, out_b.reshape(1, 3))

    return (out, hs, cs)


# speedup vs baseline: 59.7953x; 1.0722x over previous
import functools

import jax
import jax.numpy as jnp
from jax import lax
from jax.experimental import pallas as pl
from jax.experimental.pallas import tpu as pltpu
from jax.experimental.pallas import tpu_sc as plsc

HEADS = 3
NC = 2
NS = 16
NW = NC * NS


def _ale_body(ocs, ea, We0, ae0, We1, ae1, We2, ae2, o0, o1, o2):
    ea_blk = ea[...]
    eb = ea_blk.shape[0]
    for (We, ae, o, oc) in ((We0, ae0, o0, ocs[0]),
                            (We1, ae1, o1, ocs[1]),
                            (We2, ae2, o2, ocs[2])):
        cols = []
        for t in range(HEADS):
            m_t = (We[:, t * oc:(t + 1) * oc] * ae[t, :][None, :]).sum(
                axis=1, keepdims=True)
            cols.append(jnp.dot(ea_blk, m_t, preferred_element_type=jnp.float32))
        o[...] = jnp.concatenate(
            cols + [jnp.zeros((eb, 1), jnp.float32)], axis=1)


def _make_ale_kernel(E, Eb):
    grid = (E // Eb,)
    full = lambda shape: pl.BlockSpec(shape, lambda i: (0,) * len(shape))
    return pl.pallas_call(
        functools.partial(_ale_body, (16, 16, 8)),
        grid=grid,
        in_specs=[
            pl.BlockSpec((Eb, 2), lambda i: (i, 0)),
            full((2, 48)), full((3, 16)),
            full((2, 48)), full((3, 16)),
            full((2, 24)), full((3, 8)),
        ],
        out_specs=[pl.BlockSpec((Eb, 4), lambda i: (i, 0))] * 3,
        out_shape=[jax.ShapeDtypeStruct((E, 4), jnp.float32)] * 3,
    )


def _node_proj(h, a_s, a_d, oc, nb):
    hts, als_cols, ald_cols = [], [], []
    for t in range(HEADS):
        ht = h[:, t * oc:(t + 1) * oc]
        if oc < 16:
            hts.append(jnp.concatenate(
                [ht, jnp.zeros((nb, 16 - oc), jnp.float32)], axis=1))
        else:
            hts.append(ht)
        als_cols.append((ht * a_s[t, :][None, :]).sum(axis=1, keepdims=True))
        ald_cols.append((ht * a_d[t, :][None, :]).sum(axis=1, keepdims=True))
    z13 = jnp.zeros((nb, 13), jnp.float32)
    als = jnp.concatenate(als_cols + [z13], axis=1)
    ald = jnp.concatenate(ald_cols + [z13], axis=1)
    return hts, als, ald


def _kx0_body(x, W, a_s, a_d, h0, h1, h2, als, ald):
    xb = x[...]
    h = jnp.dot(xb, W[...], preferred_element_type=jnp.float32)
    hts, als_b, ald_b = _node_proj(h, a_s[...], a_d[...], 16, xb.shape[0])
    h0[...], h1[...], h2[...] = hts
    als[...], ald[...] = als_b, ald_b


def _make_kx0_kernel(N, Nb):
    grid = (N // Nb,)
    full = lambda shape: pl.BlockSpec(shape, lambda i: (0,) * len(shape))
    row = lambda w: pl.BlockSpec((Nb, w), lambda i: (i, 0))
    return pl.pallas_call(
        _kx0_body,
        grid=grid,
        in_specs=[row(3), full((3, 48)), full((3, 16)), full((3, 16))],
        out_specs=[row(16)] * 5,
        out_shape=[jax.ShapeDtypeStruct((N, 16), jnp.float32)] * 5,
    )


def _gat_epilogue(num, den, bias, oc_prev):
    outs = []
    for t in range(HEADS):
        numt = (num[t, 0] + num[t, 1])[:, :oc_prev]
        dent = (den[0, :, t] + den[1, :, t])[:, None]
        g = jnp.where(dent > 0.0, numt / dent, 0.0)
        g = g + bias[0, t * oc_prev:(t + 1) * oc_prev][None, :]
        outs.append(g)
    x = jnp.concatenate(outs, axis=1)
    return jnp.where(x > 0.0, x, jnp.exp(x) - 1.0)


def _kx12_body(oc_prev, oc, num, den, bias, W, a_s, a_d,
               h0, h1, h2, als, ald):
    x = _gat_epilogue(num[...], den[...], bias[...], oc_prev)
    h = jnp.dot(x, W[...], preferred_element_type=jnp.float32)
    hts, als_b, ald_b = _node_proj(h, a_s[...], a_d[...], oc, x.shape[0])
    h0[...], h1[...], h2[...] = hts
    als[...], ald[...] = als_b, ald_b


def _make_kx12_kernel(N, Nb, oc_prev, oc, din):
    grid = (N // Nb,)
    full = lambda shape: pl.BlockSpec(shape, lambda i: (0,) * len(shape))
    row = lambda w: pl.BlockSpec((Nb, w), lambda i: (i, 0))
    num_spec = pl.BlockSpec((3, NC, Nb, 16), lambda i: (0, 0, i, 0))
    den_spec = pl.BlockSpec((NC, Nb, 16), lambda i: (0, i, 0))
    return pl.pallas_call(
        functools.partial(_kx12_body, oc_prev, oc),
        grid=grid,
        in_specs=[num_spec, den_spec,
                  full((1, HEADS * oc_prev)), full((din, HEADS * oc)),
                  full((3, oc)), full((3, oc))],
        out_specs=[row(16)] * 5,
        out_shape=[jax.ShapeDtypeStruct((N, 16), jnp.float32)] * 5,
    )


def _tail_body(nsteps, n_total,
               num, den, bias2, meta, hn, cn,
               sp_w1, sp_b1, sp_w2, sp_b2, me_w, me_b, fc_w, fc_b,
               l0_wi, l0_wh, l0_bi, l0_bh, l1_wi, l1_wh, l1_bi, l1_bh,
               out_w, out_b, out, hs, cs, acc):
    i = pl.program_id(0)

    @pl.when(i == 0)
    def _():
        acc[...] = jnp.zeros_like(acc)

    parts = []
    num_b, den_b = num[...], den[...]
    for t in range(HEADS):
        numt = (num_b[t, 0] + num_b[t, 1])[:, :8]
        dent = (den_b[0, :, t] + den_b[1, :, t])[:, None]
        g = jnp.where(dent > 0.0, numt / dent, 0.0)
        parts.append(g.sum(axis=0, keepdims=True))
    acc[...] = acc[...] + jnp.concatenate(parts, axis=1)

    pooled = acc[...] + n_total * bias2[...]
    p = jnp.maximum(
        jnp.dot(pooled, sp_w1[...], preferred_element_type=jnp.float32)
        + sp_b1[...], 0.0)
    p = jnp.dot(p, sp_w2[...], preferred_element_type=jnp.float32) + sp_b2[...]
    m = jnp.maximum(
        jnp.dot(meta[...], me_w[...], preferred_element_type=jnp.float32)
        + me_b[...], 0.0)
    z = jnp.concatenate([p, m], axis=1)
    z = jnp.dot(z, fc_w[...], preferred_element_type=jnp.float32) + fc_b[...]
    z = jnp.where(z >= 0.0, z, 0.1 * z)

    step = z
    hs_new, cs_new = [], []
    for l, (wi, wh, bi, bh) in enumerate(
            ((l0_wi, l0_wh, l0_bi, l0_bh), (l1_wi, l1_wh, l1_bi, l1_bh))):
        g = (jnp.dot(step, wi[...], preferred_element_type=jnp.float32)
             + jnp.dot(hn[...][l:l + 1], wh[...],
                       preferred_element_type=jnp.float32)
             + bi[...] + bh[...])
        i_g, f_g, g_g, o_g = (g[:, 0:16], g[:, 16:32], g[:, 32:48], g[:, 48:64])
        c_new = (jax.nn.sigmoid(f_g) * cn[...][l:l + 1]
                 + jax.nn.sigmoid(i_g) * jnp.tanh(g_g))
        h_new = jax.nn.sigmoid(o_g) * jnp.tanh(c_new)
        hs_new.append(h_new)
        cs_new.append(c_new)
        step = h_new

    o = (jnp.dot(step, out_w[...], preferred_element_type=jnp.float32)
         + out_b[...])
    o = o - jnp.max(o, axis=1, keepdims=True)
    e = jnp.exp(o)
    out[...] = e / e.sum(axis=1, keepdims=True)
    hs[...] = jnp.concatenate(hs_new, axis=0)
    cs[...] = jnp.concatenate(cs_new, axis=0)


def _make_tail_kernel(N, Nb, n_real):
    grid = (N // Nb,)
    full = lambda shape: pl.BlockSpec(shape, lambda i: (0,) * len(shape))
    row = lambda w: pl.BlockSpec((Nb, w), lambda i: (i, 0))
    num_spec = pl.BlockSpec((3, NC, Nb, 16), lambda i: (0, 0, i, 0))
    den_spec = pl.BlockSpec((NC, Nb, 16), lambda i: (0, i, 0))
    return pl.pallas_call(
        functools.partial(_tail_body, N // Nb, n_real),
        grid=grid,
        in_specs=[num_spec, den_spec,
                  full((1, 24)), full((1, 6)), full((2, 16)), full((2, 16)),
                  full((24, 64)), full((1, 64)), full((64, 24)), full((1, 24)),
                  full((6, 8)), full((1, 8)), full((32, 16)), full((1, 16)),
                  full((16, 64)), full((16, 64)), full((1, 64)), full((1, 64)),
                  full((16, 64)), full((16, 64)), full((1, 64)), full((1, 64)),
                  full((16, 3)), full((1, 3))],
        out_specs=[full((1, 3)), full((2, 16)), full((2, 16))],
        out_shape=[jax.ShapeDtypeStruct((1, 3), jnp.float32),
                   jax.ShapeDtypeStruct((2, 16), jnp.float32),
                   jax.ShapeDtypeStruct((2, 16), jnp.float32)],
        scratch_shapes=[pltpu.VMEM((1, 24), jnp.float32)],
    )


def _make_edge_softmax_kernel(N, E, C):
    EperW = E // NW
    nch = EperW // C
    assert nch % 2 == 1
    NpT = N // NS
    nvec = C * 4 // 16
    mesh = plsc.VectorSubcoreMesh(core_axis_name="c", subcore_axis_name="s")

    @functools.partial(
        pl.kernel,
        out_type=[jax.ShapeDtypeStruct((E, 4), jnp.float32),
                  jax.ShapeDtypeStruct((NC, N, 16), jnp.float32)],
        mesh=mesh,
        compiler_params=pltpu.CompilerParams(
            use_tc_tiling_on_sc=False, needs_layout_passes=False),
        scratch_types=[pltpu.VMEM((C,), jnp.int32),
                       pltpu.VMEM((C,), jnp.int32),
                       pltpu.VMEM((C, 4), jnp.float32),
                       pltpu.VMEM((C, 16), jnp.float32),
                       pltpu.VMEM((C,), jnp.int32),
                       pltpu.VMEM((C,), jnp.int32),
                       pltpu.VMEM((C, 4), jnp.float32),
                       pltpu.VMEM((C, 16), jnp.float32),
                       pltpu.VMEM((C, 4), jnp.float32),
                       pltpu.VMEM((C, 16), jnp.float32),
                       pltpu.SemaphoreType.DMA,
                       pltpu.SemaphoreType.DMA,
                       pltpu.SemaphoreType.DMA,
                       pltpu.SemaphoreType.DMA,
                       pltpu.SemaphoreType.DMA,
                       pltpu.SemaphoreType.DMA,
                       pltpu.VMEM_SHARED((N, 16), jnp.float32)],
    )
    def k(src_hbm, dst_hbm, als_hbm, ald_hbm, ale_hbm, z16_hbm,
          ex_hbm, den_hbm,
          vsrc0, vdst0, vale0, vacc0, vsrc1, vdst1, vale1, vacc1,
          vex4, vex16, seml0, sema0, semb0, seml1, sema1, semb1, den_sh):
        cid = lax.axis_index("c")
        sid = lax.axis_index("s")
        wid = sid * NC + cid
        base = wid * EperW
        r0 = sid * NpT
        bufs = ((vsrc0, vdst0, vale0, vacc0, seml0, sema0, semb0),
                (vsrc1, vdst1, vale1, vacc1, seml1, sema1, semb1))

        pltpu.sync_copy(z16_hbm.at[pl.ds(r0, NpT)], den_sh.at[pl.ds(r0, NpT)])
        pltpu.sync_copy(z16_hbm.at[pl.ds(0, C)], vex16)
        plsc.subcore_barrier()

        iota = lax.iota(jnp.int32, 16)
        riota = iota // 4
        ciota = iota % 4

        def compute(buf):
            vsrc, vdst, vale, vacc, seml, sema, semb = buf

            def vbody(i, c2):
                for u in range(4):
                    rows = (i * 4 + u) * 4 + riota
                    v = (plsc.load_gather(vacc, [rows, ciota])
                         + plsc.load_gather(vale, [rows, ciota]))
                    v = jnp.where(v >= 0.0, v, 0.2 * v)
                    v = jnp.exp(v)
                    plsc.store_scatter(vex4, [rows, ciota], v)
                    plsc.store_scatter(vex16, [rows, ciota], v)
                return c2

            lax.fori_loop(0, nvec // 4, vbody, 0)

        pltpu.sync_copy(src_hbm.at[pl.ds(base, C)], vsrc0)
        pltpu.sync_copy(dst_hbm.at[pl.ds(base, C)], vdst0)
        pltpu.sync_copy(ale_hbm.at[pl.ds(base, C)], vale0)
        pltpu.sync_copy(als_hbm.at[vsrc0], vacc0)
        pltpu.async_copy(ald_hbm.at[vdst0], vacc0, semb0, add=True)

        def pair(kk2, carry):
            for par in range(2):
                kk = kk2 * 2 + par
                cur = bufs[par]
                nxt = bufs[1 - par]
                e0 = base + kk * C
                e0n = base + (kk + 1) * C
                pltpu.async_copy(src_hbm.at[pl.ds(e0n, C)], nxt[0], nxt[4])
                pltpu.async_copy(dst_hbm.at[pl.ds(e0n, C)], nxt[1], nxt[4])
                pltpu.async_copy(ale_hbm.at[pl.ds(e0n, C)], nxt[2], nxt[4])
                pltpu.make_async_copy(
                    ald_hbm.at[cur[1]], cur[3], cur[6]).wait()
                compute(cur)
                pltpu.make_async_copy(src_hbm.at[pl.ds(e0n, C)], nxt[0], nxt[4]).wait()
                pltpu.make_async_copy(dst_hbm.at[pl.ds(e0n, C)], nxt[1], nxt[4]).wait()
                pltpu.make_async_copy(ale_hbm.at[pl.ds(e0n, C)], nxt[2], nxt[4]).wait()
                pltpu.async_copy(als_hbm.at[nxt[0]], nxt[3], nxt[5])
                pltpu.sync_copy(vex4, ex_hbm.at[pl.ds(e0, C)])
                pltpu.make_async_copy(als_hbm.at[nxt[0]], nxt[3], nxt[5]).wait()
                pltpu.async_copy(ald_hbm.at[nxt[1]], nxt[3], nxt[6], add=True)
                pltpu.sync_copy(vex16, den_sh.at[cur[1]], add=True)
            return carry

        lax.fori_loop(0, (nch - 1) // 2, pair, 0)

        cur = bufs[0]
        e0 = base + (nch - 1) * C
        pltpu.make_async_copy(ald_hbm.at[cur[1]], cur[3], cur[6]).wait()
        compute(cur)
        pltpu.sync_copy(vex4, ex_hbm.at[pl.ds(e0, C)])
        pltpu.sync_copy(vex16, den_sh.at[cur[1]], add=True)

        plsc.subcore_barrier()
        pltpu.sync_copy(den_sh.at[pl.ds(r0, NpT)],
                        den_hbm.at[cid, pl.ds(r0, NpT)])

    return k


def _make_edge_agg_kernel(N, E, C):
    EperW = E // NW
    nch = EperW // C
    assert nch % 2 == 1
    NpT = N // NS
    mesh = plsc.VectorSubcoreMesh(core_axis_name="c", subcore_axis_name="s")

    @functools.partial(
        pl.kernel,
        out_type=[jax.ShapeDtypeStruct((3, NC, N, 16), jnp.float32)],
        mesh=mesh,
        compiler_params=pltpu.CompilerParams(
            use_tc_tiling_on_sc=False, needs_layout_passes=False),
        scratch_types=[pltpu.VMEM((C,), jnp.int32),
                       pltpu.VMEM((C,), jnp.int32),
                       pltpu.VMEM((C, 4), jnp.float32),
                       pltpu.VMEM((C, 16), jnp.float32),
                       pltpu.VMEM((C,), jnp.int32),
                       pltpu.VMEM((C,), jnp.int32),
                       pltpu.VMEM((C, 4), jnp.float32),
                       pltpu.VMEM((C, 16), jnp.float32),
                       pltpu.SemaphoreType.DMA,
                       pltpu.SemaphoreType.DMA,
                       pltpu.SemaphoreType.DMA,
                       pltpu.SemaphoreType.DMA,
                       pltpu.VMEM_SHARED((N, 16), jnp.float32)],
    )
    def k(src_hbm, dst_hbm, ex_hbm, h0_hbm, h1_hbm, h2_hbm, z16_hbm,
          num_hbm, vsrc0, vdst0, vex0, vh0, vsrc1, vdst1, vex1, vh1,
          seml0, semg0, seml1, semg1, num_sh):
        cid = lax.axis_index("c")
        sid = lax.axis_index("s")
        wid = sid * NC + cid
        base = wid * EperW
        r0 = sid * NpT
        bufs = ((vsrc0, vdst0, vex0, vh0, seml0, semg0),
                (vsrc1, vdst1, vex1, vh1, seml1, semg1))

        def load_linear(e0, buf, sync):
            vsrc, vdst, vex, vh, seml, semg = buf
            if sync:
                pltpu.sync_copy(src_hbm.at[pl.ds(e0, C)], vsrc)
                pltpu.sync_copy(dst_hbm.at[pl.ds(e0, C)], vdst)
                pltpu.sync_copy(ex_hbm.at[pl.ds(e0, C)], vex)
            else:
                pltpu.async_copy(src_hbm.at[pl.ds(e0, C)], vsrc, seml)
                pltpu.async_copy(dst_hbm.at[pl.ds(e0, C)], vdst, seml)
                pltpu.async_copy(ex_hbm.at[pl.ds(e0, C)], vex, seml)

        def wait_linear(e0, buf):
            vsrc, vdst, vex, vh, seml, semg = buf
            pltpu.make_async_copy(src_hbm.at[pl.ds(e0, C)], vsrc, seml).wait()
            pltpu.make_async_copy(dst_hbm.at[pl.ds(e0, C)], vdst, seml).wait()
            pltpu.make_async_copy(ex_hbm.at[pl.ds(e0, C)], vex, seml).wait()

        for t in range(HEADS):
            h_hbm = (h0_hbm, h1_hbm, h2_hbm)[t]
            pltpu.sync_copy(z16_hbm.at[pl.ds(r0, NpT)],
                            num_sh.at[pl.ds(r0, NpT)])
            plsc.subcore_barrier()

            tvec = jnp.full((16,), t, jnp.int32)

            def compute(buf):
                vsrc, vdst, vex, vh, seml, semg = buf

                def ebody(c2, cc):
                    for u in range(8):
                        c3 = c2 * 8 + u
                        sc = plsc.load_gather(
                            vex, [jnp.full((16,), c3, jnp.int32), tvec])
                        vh[c3, :] = vh[c3, :] * sc
                    return cc

                lax.fori_loop(0, C // 8, ebody, 0)

            load_linear(base, bufs[0], True)
            pltpu.async_copy(h_hbm.at[vsrc0], vh0, semg0)

            def pair(kk2, carry):
                for par in range(2):
                    kk = kk2 * 2 + par
                    cur = bufs[par]
                    nxt = bufs[1 - par]
                    e0n = base + (kk + 1) * C
                    load_linear(e0n, nxt, False)
                    pltpu.make_async_copy(
                        h_hbm.at[cur[0]], cur[3], cur[5]).wait()
                    compute(cur)
                    wait_linear(e0n, nxt)
                    pltpu.async_copy(h_hbm.at[nxt[0]], nxt[3], nxt[5])
                    pltpu.sync_copy(cur[3], num_sh.at[cur[1]], add=True)
                return carry

            lax.fori_loop(0, (nch - 1) // 2, pair, 0)

            cur = bufs[0]
            pltpu.make_async_copy(h_hbm.at[cur[0]], cur[3], cur[5]).wait()
            compute(cur)
            pltpu.sync_copy(cur[3], num_sh.at[cur[1]], add=True)

            plsc.subcore_barrier()
            pltpu.sync_copy(num_sh.at[pl.ds(r0, NpT)],
                            num_hbm.at[t, cid, pl.ds(r0, NpT)])
            plsc.subcore_barrier()

    return k


def kernel(x, edge_index, edge_attr, meta_data, hn, cn,
           W0, We0, asrc0, adst0, aedge0, bias0,
           W1, We1, asrc1, adst1, aedge1, bias1,
           W2, We2, asrc2, adst2, aedge2, bias2,
           sp_w1, sp_b1, sp_w2, sp_b2, me_w, me_b, fc_w, fc_b,
           l0_wi, l0_wh, l0_bi, l0_bh, l1_wi, l1_wh, l1_bi, l1_bh,
           out_w, out_b):
    N = x.shape[0]
    E = edge_index.shape[1]
    N2 = ((N + 2047) // 2048) * 2048
    CA = 400
    CE = 400
    Nb = 2048
    Eb = 4000

    src = edge_index[0]
    dst = edge_index[1]
    xp = jnp.pad(x, ((0, N2 - N), (0, 0)))
    z16 = jnp.zeros((N2, 16), jnp.float32)

    ale0, ale1, ale2 = _make_ale_kernel(E, Eb)(
        edge_attr, We0, aedge0, We1, aedge1, We2, aedge2)

    edge_softmax = _make_edge_softmax_kernel(N2, E, CA)
    edge_agg = _make_edge_agg_kernel(N2, E, CE)

    h0, h1, h2, als, ald = _make_kx0_kernel(N2, Nb)(xp, W0, asrc0, adst0)
    ex, den = edge_softmax(src, dst, als, ald, ale0, z16)
    num, = edge_agg(src, dst, ex, h0, h1, h2, z16)

    h0, h1, h2, als, ald = _make_kx12_kernel(N2, Nb, 16, 16, 48)(
        num, den, bias0.reshape(1, 48), W1, asrc1, adst1)
    ex, den = edge_softmax(src, dst, als, ald, ale1, z16)
    num, = edge_agg(src, dst, ex, h0, h1, h2, z16)

    h0, h1, h2, als, ald = _make_kx12_kernel(N2, Nb, 16, 8, 48)(
        num, den, bias1.reshape(1, 48), W2, asrc2, adst2)
    ex, den = edge_softmax(src, dst, als, ald, ale2, z16)
    num, = edge_agg(src, dst, ex, h0, h1, h2, z16)

    out, hs, cs = _make_tail_kernel(N2, Nb, float(N))(
        num, den,
        bias2.reshape(1, 24), meta_data.reshape(1, 6), hn, cn,
        sp_w1, sp_b1.reshape(1, 64), sp_w2, sp_b2.reshape(1, 24),
        me_w, me_b.reshape(1, 8), fc_w, fc_b.reshape(1, 16),
        l0_wi, l0_wh, l0_bi.reshape(1, 64), l0_bh.reshape(1, 64),
        l1_wi, l1_wh, l1_bi.reshape(1, 64), l1_bh.reshape(1, 64),
        out_w, out_b.reshape(1, 3))

    return (out, hs, cs)
